# baseline (speedup 1.0000x reference)
import functools

import jax
import jax.numpy as jnp
from jax import lax
from jax.experimental import pallas as pl
from jax.experimental.pallas import tpu as pltpu
from jax.experimental.pallas import tpu_sc as plsc

_L = 17
_JL = 8


@functools.lru_cache(maxsize=None)
def _build_lut_gather(n_rows: int):
    info = plsc.get_sparse_core_info()
    nc, ns = info.num_cores, info.num_subcores
    nw = nc * ns
    chunk = _JL * 128
    assert n_rows % (nw * chunk) == 0
    jb = n_rows // 128
    jw = jb // nw
    n_chunks = jw // _JL
    out_plane = jb * 512

    mesh = plsc.VectorSubcoreMesh(core_axis_name="c", subcore_axis_name="s")

    @functools.partial(
        pl.kernel,
        mesh=mesh,
        out_type=jax.ShapeDtypeStruct((4 * out_plane,), jnp.float32),
        scratch_types=[
            pltpu.VMEM((chunk * 4,), jnp.int32),
            pltpu.VMEM((chunk,), jnp.int32),
            pltpu.VMEM((chunk, 16), jnp.float32),
            pltpu.VMEM((chunk * 4,), jnp.float32),
            pltpu.VMEM((chunk * 4,), jnp.float32),
            pltpu.VMEM((chunk * 4,), jnp.float32),
            pltpu.VMEM((chunk * 4,), jnp.float32),
            pltpu.SemaphoreType.DMA,
        ],
        compiler_params=pltpu.CompilerParams(
            needs_layout_passes=False, use_tc_tiling_on_sc=False
        ),
    )
    def lut_gather(table_hbm, idx_hbm, out_hbm, idxc, flat, rows,
                   t0, t1, t2, t3, sem):
        wid = lax.axis_index("s") * nc + lax.axis_index("c")
        base_j = wid * jw
        trsp = (t0, t1, t2, t3)

        cols = [jnp.full((16,), c, jnp.int32) for c in range(16)]
        lanes = [k * 16 + lax.iota(jnp.int32, 16) for k in range(8)]

        def chunk_body(t, carry):
            j0 = base_j + t * _JL
            pltpu.sync_copy(idx_hbm.at[pl.ds(j0 * 512, chunk * 4)], idxc)

            def flat_body(jl, carry2):
                s = jl * 512
                o = jl * 128
                for k in range(8):
                    i0 = idxc[pl.ds(s + k * 16, 16)]
                    i1 = idxc[pl.ds(s + 128 + k * 16, 16)]
                    i2 = idxc[pl.ds(s + 256 + k * 16, 16)]
                    i3 = idxc[pl.ds(s + 384 + k * 16, 16)]
                    f = ((i0 * _L + i1) * _L + i2) * _L + i3
                    flat[pl.ds(o + k * 16, 16)] = f
                return carry2

            lax.fori_loop(0, _JL, flat_body, 0)
            pltpu.async_copy(table_hbm.at[flat], rows, sem).wait()

            def tr_body(jl, carry2):
                rb = jl * 128
                sb = jl * 512
                for k in range(8):
                    r = rb + lanes[k]
                    for u in range(4):
                        for v in range(4):
                            val = plsc.load_gather(rows, [r, cols[4 * u + v]])
                            trsp[u][pl.ds(sb + v * 128 + k * 16, 16)] = val
                return carry2

            lax.fori_loop(0, _JL, tr_body, 0)

            for u in range(4):
                pltpu.sync_copy(
                    trsp[u],
                    out_hbm.at[pl.ds(u * out_plane + j0 * 512, chunk * 4)],
                )
            return carry

        lax.fori_loop(0, n_chunks, chunk_body, 0)

    return lut_gather


def kernel(weight, index):
    n = index.shape[0]
    up = weight.shape[-1]
    table = weight.reshape(-1, up * up)
    idxv = (
        index.astype(jnp.int32)
        .reshape(n // 128, 128, 4)
        .transpose(0, 2, 1)
        .reshape(n * 4)
    )
    o = _build_lut_gather(n)(table, idxv)
    return (
        o.reshape(4, n // 128, 4, 128)
        .transpose(1, 3, 0, 2)
        .reshape(n, up, up)
    )


# speedup vs baseline: 2.1402x; 2.1402x over previous
import functools

import jax
import jax.numpy as jnp
from jax import lax
from jax.experimental import pallas as pl
from jax.experimental.pallas import tpu as pltpu
from jax.experimental.pallas import tpu_sc as plsc

_L = 17
_JL = 8


@functools.lru_cache(maxsize=None)
def _build_lut_gather(n_rows: int):
    info = plsc.get_sparse_core_info()
    nc, ns = info.num_cores, info.num_subcores
    nw = nc * ns
    chunk = _JL * 128
    assert n_rows % (nw * chunk) == 0
    jb = n_rows // 128
    jw = jb // nw
    n_chunks = jw // _JL
    assert n_chunks % 2 == 0 and n_chunks >= 4
    out_plane = jb * 512
    idx_len = n_rows * 4

    mesh = plsc.VectorSubcoreMesh(core_axis_name="c", subcore_axis_name="s")

    @functools.partial(
        pl.kernel,
        mesh=mesh,
        out_type=jax.ShapeDtypeStruct((4 * out_plane,), jnp.float32),
        scratch_types=[
            pltpu.VMEM((2, chunk * 4), jnp.int32),
            pltpu.VMEM((2, chunk), jnp.int32),
            pltpu.VMEM((2, chunk, 16), jnp.float32),
            pltpu.VMEM((2, 4 * chunk * 4), jnp.float32),
            pltpu.SemaphoreType.DMA,
            pltpu.SemaphoreType.DMA,
            pltpu.SemaphoreType.DMA,
            pltpu.SemaphoreType.DMA,
            pltpu.SemaphoreType.DMA,
            pltpu.SemaphoreType.DMA,
        ],
        compiler_params=pltpu.CompilerParams(
            needs_layout_passes=False, use_tc_tiling_on_sc=False
        ),
    )
    def lut_gather(table_hbm, idx_hbm, out_hbm, idxc2, flat2, rows2, trsp2,
                   si0, si1, sg0, sg1, so0, so1):
        wid = lax.axis_index("s") * nc + lax.axis_index("c")
        base_j = wid * jw
        sem_i = (si0, si1)
        sem_g = (sg0, sg1)
        sem_o = (so0, so1)

        iota = lax.iota(jnp.int32, 16)
        diag_c = [(iota + d) & 15 for d in range(16)]
        diag_w = [
            ((diag_c[d] >> 2) * (chunk * 4)) + ((diag_c[d] & 3) * 128) + iota
            for d in range(16)
        ]

        def idx_off(t):
            return jnp.minimum((base_j + t * _JL) * 512, idx_len - chunk * 4)

        def issue_idx(t, p):
            pltpu.async_copy(
                idx_hbm.at[pl.ds(idx_off(t), chunk * 4)], idxc2.at[p], sem_i[p]
            )

        def wait_idx(p):
            pltpu.make_async_copy(
                idx_hbm.at[pl.ds(0, chunk * 4)], idxc2.at[p], sem_i[p]
            ).wait()

        def flatten(p):
            idxc = idxc2.at[p]
            flat = flat2.at[p]

            def body(jl, carry):
                s = jl * 512
                o = jl * 128
                for k in range(8):
                    i0 = idxc[pl.ds(s + k * 16, 16)]
                    i1 = idxc[pl.ds(s + 128 + k * 16, 16)]
                    i2 = idxc[pl.ds(s + 256 + k * 16, 16)]
                    i3 = idxc[pl.ds(s + 384 + k * 16, 16)]
                    f = ((i0 * _L + i1) * _L + i2) * _L + i3
                    flat[pl.ds(o + k * 16, 16)] = f
                return carry

            lax.fori_loop(0, _JL, body, 0)

        def issue_gather(p):
            pltpu.async_copy(
                table_hbm.at[flat2.at[p]], rows2.at[p], sem_g[p]
            )

        def wait_gather(p):
            pltpu.make_async_copy(
                table_hbm.at[flat2.at[p]], rows2.at[p], sem_g[p]
            ).wait()

        def transpose(p):
            rows = rows2.at[p]
            trsp = trsp2.at[p]

            def body(jl, carry):
                for k in range(8):
                    rvec = (jl * 128 + k * 16) + iota
                    wb = jl * 512 + k * 16
                    for d in range(16):
                        val = plsc.load_gather(rows, [rvec, diag_c[d]])
                        plsc.store_scatter(trsp, [wb + diag_w[d]], val)
                return carry

            lax.fori_loop(0, _JL, body, 0)

        def issue_out(t, p):
            for u in range(4):
                pltpu.async_copy(
                    trsp2.at[p, pl.ds(u * chunk * 4, chunk * 4)],
                    out_hbm.at[
                        pl.ds(u * out_plane + (base_j + t * _JL) * 512, chunk * 4)
                    ],
                    sem_o[p],
                )

        def wait_out(p):
            for u in range(4):
                pltpu.make_async_copy(
                    out_hbm.at[pl.ds(0, chunk * 4)],
                    trsp2.at[p, pl.ds(u * chunk * 4, chunk * 4)],
                    sem_o[p],
                ).wait()

        def chunk_step(t, p, prefetch):
            wait_idx(p)
            flatten(p)
            issue_gather(p)
            if prefetch:
                issue_idx(t + 1, 1 - p)
            wait_gather(1 - p)
            wait_out(1 - p)
            transpose(1 - p)
            issue_out(t - 1, 1 - p)

        for q in range(2):
            for u in range(4):
                pltpu.async_copy(
                    out_hbm.at[pl.ds(0, chunk * 4)],
                    trsp2.at[q, pl.ds(u * chunk * 4, chunk * 4)],
                    sem_o[q],
                )

        pltpu.sync_copy(idx_hbm.at[pl.ds(idx_off(0), chunk * 4)], idxc2.at[0])
        flatten(0)
        issue_gather(0)
        issue_idx(1, 1)

        def pair_body(g, carry):
            chunk_step(2 * g + 1, 1, True)
            chunk_step(2 * g + 2, 0, True)
            return carry

        lax.fori_loop(0, (n_chunks - 2) // 2, pair_body, 0)

        t_last = n_chunks - 1
        chunk_step(t_last, 1, False)

        wait_gather(1)
        wait_out(1)
        transpose(1)
        issue_out(t_last, 1)
        wait_out(0)
        wait_out(1)

    return lut_gather


def kernel(weight, index):
    n = index.shape[0]
    up = weight.shape[-1]
    table = weight.reshape(-1, up * up)
    idxv = (
        index.astype(jnp.int32)
        .reshape(n // 128, 128, 4)
        .transpose(0, 2, 1)
        .reshape(n * 4)
    )
    o = _build_lut_gather(n)(table, idxv)
    return (
        o.reshape(4, n // 128, 4, 128)
        .transpose(1, 3, 0, 2)
        .reshape(n, up, up)
    )


# speedup vs baseline: 2.2024x; 1.0290x over previous
import functools

import jax
import jax.numpy as jnp
from jax import lax
from jax.experimental import pallas as pl
from jax.experimental.pallas import tpu as pltpu
from jax.experimental.pallas import tpu_sc as plsc

_L = 17
_JL = 8


@functools.lru_cache(maxsize=None)
def _build_lut_gather(n_rows: int):
    info = plsc.get_sparse_core_info()
    nc, ns = info.num_cores, info.num_subcores
    nw = nc * ns
    chunk = _JL * 128
    assert n_rows % (nw * chunk) == 0
    jb = n_rows // 128
    jw = jb // nw
    n_chunks = jw // _JL
    assert n_chunks % 2 == 0 and n_chunks >= 4
    out_plane = jb * 512
    idx_len = n_rows * 4

    mesh = plsc.VectorSubcoreMesh(core_axis_name="c", subcore_axis_name="s")

    @functools.partial(
        pl.kernel,
        mesh=mesh,
        out_type=jax.ShapeDtypeStruct((4 * out_plane,), jnp.float32),
        scratch_types=[
            pltpu.VMEM((2, chunk * 4), jnp.int32),
            pltpu.VMEM((2, chunk), jnp.int32),
            pltpu.VMEM((2, chunk, 16), jnp.float32),
            pltpu.VMEM((2, 4 * chunk * 4 + 16), jnp.float32),
            pltpu.SemaphoreType.DMA,
            pltpu.SemaphoreType.DMA,
            pltpu.SemaphoreType.DMA,
            pltpu.SemaphoreType.DMA,
            pltpu.SemaphoreType.DMA,
            pltpu.SemaphoreType.DMA,
        ],
        compiler_params=pltpu.CompilerParams(
            needs_layout_passes=False, use_tc_tiling_on_sc=False
        ),
    )
    def lut_gather(table_hbm, idx_hbm, out_hbm, idxc2, flat2, rows2,
                   trsp2, si0, si1, sg0, sg1, so0, so1):
        wid = lax.axis_index("s") * nc + lax.axis_index("c")
        base_j = wid * jw
        sem_i = (si0, si1)
        sem_g = (sg0, sg1)
        sem_o = (so0, so1)

        iota = lax.iota(jnp.int32, 16)
        diag_c = [(iota + d) & 15 for d in range(16)]
        diag_w = [
            ((diag_c[d] >> 2) * (chunk * 4)) + ((diag_c[d] & 3) * 128) + iota
            for d in range(16)
        ]

        def idx_off(t):
            return jnp.minimum((base_j + t * _JL) * 512, idx_len - chunk * 4)

        def issue_idx(t, p):
            pltpu.async_copy(
                idx_hbm.at[pl.ds(idx_off(t), chunk * 4)], idxc2.at[p], sem_i[p]
            )

        def wait_idx(p):
            pltpu.make_async_copy(
                idx_hbm.at[pl.ds(0, chunk * 4)], idxc2.at[p], sem_i[p]
            ).wait()

        def flatten(p):
            idxc = idxc2.at[p]
            flat = flat2.at[p]

            def body(jl, carry):
                s = jl * 512
                o = jl * 128
                for k in range(8):
                    i0 = idxc[pl.ds(s + k * 16, 16)]
                    i1 = idxc[pl.ds(s + 128 + k * 16, 16)]
                    i2 = idxc[pl.ds(s + 256 + k * 16, 16)]
                    i3 = idxc[pl.ds(s + 384 + k * 16, 16)]
                    f = ((i0 * _L + i1) * _L + i2) * _L + i3
                    flat[pl.ds(o + k * 16, 16)] = f
                return carry

            lax.fori_loop(0, _JL, body, 0)

        def issue_gather(p):
            pltpu.async_copy(
                table_hbm.at[flat2.at[p]], rows2.at[p], sem_g[p]
            )

        def wait_gather(p):
            pltpu.make_async_copy(
                table_hbm.at[flat2.at[p]], rows2.at[p], sem_g[p]
            ).wait()

        def transpose(p):
            w_span = 3 * chunk * 4 + 3 * 128 + 16

            def body(jl, carry):
                for k in range(8):
                    rb = jl * 128 + k * 16
                    wb = jl * 512 + k * 16
                    rblk = rows2.at[p, pl.ds(rb, 16)]
                    tblk = trsp2.at[p, pl.ds(wb, w_span)]
                    for d in range(16):
                        val = plsc.load_gather(rblk, [iota, diag_c[d]])
                        plsc.store_scatter(tblk, [diag_w[d]], val)
                return carry

            lax.fori_loop(0, _JL, body, 0)

        def issue_out(t, p):
            for u in range(4):
                pltpu.async_copy(
                    trsp2.at[p, pl.ds(u * chunk * 4, chunk * 4)],
                    out_hbm.at[
                        pl.ds(u * out_plane + (base_j + t * _JL) * 512, chunk * 4)
                    ],
                    sem_o[p],
                )

        def wait_out(p):
            for u in range(4):
                pltpu.make_async_copy(
                    out_hbm.at[pl.ds(0, chunk * 4)],
                    trsp2.at[p, pl.ds(u * chunk * 4, chunk * 4)],
                    sem_o[p],
                ).wait()

        def chunk_step(t, p, prefetch):
            wait_idx(p)
            flatten(p)
            issue_gather(p)
            if prefetch:
                issue_idx(t + 1, 1 - p)
            wait_gather(1 - p)
            wait_out(1 - p)
            transpose(1 - p)
            issue_out(t - 1, 1 - p)

        for q in range(2):
            for u in range(4):
                pltpu.async_copy(
                    out_hbm.at[pl.ds(0, chunk * 4)],
                    trsp2.at[q, pl.ds(u * chunk * 4, chunk * 4)],
                    sem_o[q],
                )

        pltpu.sync_copy(idx_hbm.at[pl.ds(idx_off(0), chunk * 4)], idxc2.at[0])
        flatten(0)
        issue_gather(0)
        issue_idx(1, 1)

        def pair_body(g, carry):
            chunk_step(2 * g + 1, 1, True)
            chunk_step(2 * g + 2, 0, True)
            return carry

        lax.fori_loop(0, (n_chunks - 2) // 2, pair_body, 0)

        t_last = n_chunks - 1
        chunk_step(t_last, 1, False)

        wait_gather(1)
        wait_out(1)
        transpose(1)
        issue_out(t_last, 1)
        wait_out(0)
        wait_out(1)

    return lut_gather


def kernel(weight, index):
    n = index.shape[0]
    up = weight.shape[-1]
    table = weight.reshape(-1, up * up)
    idxv = (
        index.astype(jnp.int32)
        .reshape(n // 128, 128, 4)
        .transpose(0, 2, 1)
        .reshape(n * 4)
    )
    o = _build_lut_gather(n)(table, idxv)
    return (
        o.reshape(4, n // 128, 4, 128)
        .transpose(1, 3, 0, 2)
        .reshape(n, up, up)
    )
